# baseline (speedup 1.0000x reference)
import functools

import jax
import jax.numpy as jnp
from jax import lax
from jax.experimental import pallas as pl
from jax.experimental.pallas import tpu as pltpu
from jax.experimental.pallas import tpu_sc as plsc

NC = 2
NS = 16
NW = NC * NS
L = 16

CHUNK = 128


def _sc_body(B, D, b_per_w,
             user_idx_hbm, item_idx_hbm, know_hbm,
             theta_tab_hbm, a_tab_hbm, b_tab_hbm,
             pred_out, theta_out, a_out, b_out,
             uidx_v, iidx_v, uidx_h, iidx_h,
             theta_p, a_p, know_p, theta_c, a_c, b_v, pred_v,
             sem_g, sem_o):
    n_chunks = b_per_w // CHUNK
    wid = lax.axis_index("s") * NC + lax.axis_index("c")
    base = wid * b_per_w

    idx_copies = []
    for j in range(n_chunks):
        idx_copies.append(pltpu.async_copy(
            user_idx_hbm.at[pl.ds(base + j * CHUNK, CHUNK)],
            uidx_v.at[j], sem_g))
        idx_copies.append(pltpu.async_copy(
            item_idx_hbm.at[pl.ds(base + j * CHUNK, CHUNK)],
            iidx_v.at[j], sem_g))
    for c in idx_copies:
        c.wait()

    for j in range(n_chunks):
        for k in range(CHUNK // L):
            sl = pl.ds(k * L, L)
            uidx_h[j, sl] = lax.shift_right_logical(uidx_v[j, sl], 1)
            iidx_h[j, sl] = lax.shift_right_logical(iidx_v[j, sl], 1)

    lane = lax.iota(jnp.int32, L)
    n_sub = D // L
    out_copies = []

    for j in range(n_chunks):
        gathers = [
            pltpu.async_copy(theta_tab_hbm.at[uidx_h.at[j]], theta_p, sem_g),
            pltpu.async_copy(a_tab_hbm.at[iidx_h.at[j]], a_p, sem_g),
            pltpu.async_copy(b_tab_hbm.at[iidx_v.at[j]],
                             b_v.at[pl.ds(j * CHUNK, CHUNK)], sem_g),
            pltpu.async_copy(know_hbm.at[pl.ds(base + j * CHUNK, CHUNK)],
                             know_p, sem_g),
        ]
        for c in gathers:
            c.wait()

        def group_body(g, carry, j=j):
            u16 = uidx_v[j, pl.ds(g * L, L)]
            i16 = iidx_v[j, pl.ds(g * L, L)]
            toff16 = lax.shift_left(jnp.bitwise_and(u16, 1), 6)
            aoff16 = lax.shift_left(jnp.bitwise_and(i16, 1), 6)
            zvec = jnp.zeros((L,), jnp.float32)
            for rl in range(L):
                r_loc = g * L + rl
                R = j * CHUNK + r_loc
                toff = toff16[rl]
                aoff = aoff16[rl]
                acc = None
                for c in range(n_sub):
                    tv = theta_p[r_loc, pl.ds(toff + c * L, L)]
                    av = a_p[r_loc, pl.ds(aoff + c * L, L)]
                    kv = know_p[r_loc, pl.ds(c * L, L)]
                    theta_c[R, pl.ds(c * L, L)] = tv
                    a_c[R, pl.ds(c * L, L)] = av
                    prod = tv * av * kv
                    acc = prod if acc is None else acc + prod
                zvec = jnp.where(lane == rl, jnp.sum(acc), zvec)
            z = zvec - b_v[pl.ds(j * CHUNK + g * L, L)]
            pred_v[pl.ds(j * CHUNK + g * L, L)] = 1.0 / (1.0 + jnp.exp(-z))
            return carry

        lax.fori_loop(0, CHUNK // L, group_body, 0)

    out_copies = [
        pltpu.async_copy(theta_c, theta_out.at[pl.ds(base, b_per_w)], sem_o),
        pltpu.async_copy(a_c, a_out.at[pl.ds(base, b_per_w)], sem_o),
        pltpu.async_copy(b_v, b_out.at[pl.ds(base, b_per_w)], sem_o),
        pltpu.async_copy(pred_v, pred_out.at[pl.ds(base, b_per_w)], sem_o),
    ]
    for c in out_copies:
        c.wait()


@jax.jit
def _emb_icd(user_idx, item_idx, know, theta_table, a_table, b_table):
    B, D = know.shape
    assert B % (NW * CHUNK) == 0 and D % L == 0
    b_per_w = B // NW
    n_chunks = b_per_w // CHUNK

    mesh = plsc.VectorSubcoreMesh(core_axis_name="c", subcore_axis_name="s",
                                  num_cores=NC, num_subcores=NS)
    fn = pl.kernel(
        functools.partial(_sc_body, B, D, b_per_w),
        out_type=(
            jax.ShapeDtypeStruct((B,), jnp.float32),
            jax.ShapeDtypeStruct((B, D), jnp.float32),
            jax.ShapeDtypeStruct((B, D), jnp.float32),
            jax.ShapeDtypeStruct((B,), jnp.float32),
        ),
        mesh=mesh,
        scratch_types=[
            pltpu.VMEM((n_chunks, CHUNK), jnp.int32),
            pltpu.VMEM((n_chunks, CHUNK), jnp.int32),
            pltpu.VMEM((n_chunks, CHUNK), jnp.int32),
            pltpu.VMEM((n_chunks, CHUNK), jnp.int32),
            pltpu.VMEM((CHUNK, 2 * D), jnp.float32),
            pltpu.VMEM((CHUNK, 2 * D), jnp.float32),
            pltpu.VMEM((CHUNK, D), jnp.float32),
            pltpu.VMEM((b_per_w, D), jnp.float32),
            pltpu.VMEM((b_per_w, D), jnp.float32),
            pltpu.VMEM((b_per_w,), jnp.float32),
            pltpu.VMEM((b_per_w,), jnp.float32),
            pltpu.SemaphoreType.DMA,
            pltpu.SemaphoreType.DMA,
        ],
        compiler_params=pltpu.CompilerParams(needs_layout_passes=False,
                                             use_tc_tiling_on_sc=False),
        name="emb_icd_sc",
    )
    theta_pairs = theta_table.reshape(-1, 2 * D)
    a_pairs = a_table.reshape(-1, 2 * D)
    return fn(user_idx, item_idx, know, theta_pairs, a_pairs,
              b_table.reshape(-1))


def kernel(user_idx, item_idx, know, theta_table, a_table, b_table):
    user_idx = user_idx.astype(jnp.int32)
    item_idx = item_idx.astype(jnp.int32)
    pred, theta, a, b_flat = _emb_icd(user_idx, item_idx, know,
                                      theta_table, a_table, b_table)
    return (pred, theta, a, b_flat.reshape(-1, 1))


# speedup vs baseline: 1.9517x; 1.9517x over previous
import functools

import jax
import jax.numpy as jnp
from jax import lax
from jax.experimental import pallas as pl
from jax.experimental.pallas import tpu as pltpu
from jax.experimental.pallas import tpu_sc as plsc

NC = 2
NS = 16
NW = NC * NS
L = 16
TS = 8

CHUNK = 16


def _sc_body(B, D, b_per_w,
             user_idx_hbm, item_idx_hbm, know_hbm,
             theta_tab_hbm, a_tab_hbm, b_tab_hbm,
             pred_out, theta_out, a_out, b_out,
             uidx_v, iidx_v, utile_v, itile_v,
             theta_p0, theta_p1, a_p0, a_p1, know_p0, know_p1,
             t_cc0, t_cc1, a_cc0, a_cc1, b_v, pred_v,
             sem_g0, sem_g1, sem_o0, sem_o1):
    n_chunks = b_per_w // CHUNK
    wid = lax.axis_index("s") * NC + lax.axis_index("c")
    base = wid * b_per_w

    pltpu.sync_copy(user_idx_hbm.at[pl.ds(base, b_per_w)], uidx_v)
    pltpu.sync_copy(item_idx_hbm.at[pl.ds(base, b_per_w)], iidx_v)
    for k in range(b_per_w // L):
        sl = pl.ds(k * L, L)
        utile_v[sl] = lax.shift_right_logical(uidx_v[sl], 3)
        itile_v[sl] = lax.shift_right_logical(iidx_v[sl], 3)

    theta_p = (theta_p0, theta_p1)
    a_p = (a_p0, a_p1)
    know_p = (know_p0, know_p1)
    t_cc = (t_cc0, t_cc1)
    a_cc = (a_cc0, a_cc1)
    sem_g = (sem_g0, sem_g1)
    sem_o = (sem_o0, sem_o1)

    def fire_gathers(cix, b):
        isl = pl.ds(cix * CHUNK, CHUNK)
        ut16 = utile_v[isl]
        it16 = itile_v[isl]
        for rl in range(CHUNK):
            pltpu.async_copy(theta_tab_hbm.at[ut16[rl]], theta_p[b].at[rl],
                             sem_g[b])
            pltpu.async_copy(a_tab_hbm.at[it16[rl]], a_p[b].at[rl],
                             sem_g[b])
        pltpu.async_copy(b_tab_hbm.at[iidx_v.at[isl]], b_v.at[isl], sem_g[b])
        pltpu.async_copy(know_hbm.at[pl.ds((base + cix * CHUNK) * D,
                                           CHUNK * D)],
                         know_p[b], sem_g[b])

    def wait_gathers(cix, b):
        isl = pl.ds(cix * CHUNK, CHUNK)
        for rl in range(CHUNK):
            pltpu.make_async_copy(theta_tab_hbm.at[0], theta_p[b].at[rl],
                                  sem_g[b]).wait()
            pltpu.make_async_copy(a_tab_hbm.at[0], a_p[b].at[rl],
                                  sem_g[b]).wait()
        pltpu.make_async_copy(b_tab_hbm.at[iidx_v.at[isl]], b_v.at[isl],
                              sem_g[b]).wait()
        pltpu.make_async_copy(know_hbm.at[pl.ds((base + cix * CHUNK) * D,
                                                CHUNK * D)],
                              know_p[b], sem_g[b]).wait()

    def fire_out(cix, b):
        osl = pl.ds((base + cix * CHUNK) * D, CHUNK * D)
        pltpu.async_copy(t_cc[b], theta_out.at[osl], sem_o[b])
        pltpu.async_copy(a_cc[b], a_out.at[osl], sem_o[b])

    def wait_out(cix, b):
        osl = pl.ds((base + cix * CHUNK) * D, CHUNK * D)
        pltpu.make_async_copy(t_cc[b], theta_out.at[osl], sem_o[b]).wait()
        pltpu.make_async_copy(a_cc[b], a_out.at[osl], sem_o[b]).wait()

    lane = lax.iota(jnp.int32, L)
    n_sub = D // L

    def compute(cix, b):
        isl = pl.ds(cix * CHUNK, CHUNK)
        u16 = uidx_v[isl]
        i16 = iidx_v[isl]
        su16 = jnp.bitwise_and(u16, TS - 1)
        si16 = jnp.bitwise_and(i16, TS - 1)
        zvec = jnp.zeros((L,), jnp.float32)
        for rl in range(L):
            su = su16[rl]
            si = si16[rl]
            acc = None
            for c in range(n_sub):
                csl = pl.ds(c * L, L)
                tv = theta_p[b][rl, su, csl]
                av = a_p[b][rl, si, csl]
                kv = know_p[b][pl.ds(rl * D + c * L, L)]
                t_cc[b][pl.ds(rl * D + c * L, L)] = tv
                a_cc[b][pl.ds(rl * D + c * L, L)] = av
                prod = tv * av * kv
                acc = prod if acc is None else acc + prod
            zvec = jnp.where(lane == rl, jnp.sum(acc), zvec)
        z = zvec - b_v[isl]
        pred_v[isl] = 1.0 / (1.0 + jnp.exp(-z))

    fire_gathers(0, 0)

    def loop_body(jj, carry):
        for b in (0, 1):
            cix = jj * 2 + b
            wait_gathers(cix, b)
            nxt = jnp.minimum(cix + 1, n_chunks - 1)
            fire_gathers(nxt, 1 - b)

            @pl.when(jj >= 1)
            def _():
                wait_out(cix, b)

            compute(cix, b)
            fire_out(cix, b)
        return carry

    lax.fori_loop(0, n_chunks // 2, loop_body, 0)

    wait_gathers(n_chunks - 1, 0)
    wait_out(n_chunks - 2, 0)
    wait_out(n_chunks - 1, 1)

    pltpu.async_copy(pred_v, pred_out.at[pl.ds(base, b_per_w)], sem_o0)
    pltpu.async_copy(b_v, b_out.at[pl.ds(base, b_per_w)], sem_o0)
    pltpu.make_async_copy(pred_v, pred_out.at[pl.ds(base, b_per_w)],
                          sem_o0).wait()
    pltpu.make_async_copy(b_v, b_out.at[pl.ds(base, b_per_w)],
                          sem_o0).wait()


@jax.jit
def _emb_icd(user_idx, item_idx, know, theta_table, a_table, b_table):
    B, D = know.shape
    assert B % (NW * CHUNK) == 0 and D % L == 0
    b_per_w = B // NW

    mesh = plsc.VectorSubcoreMesh(core_axis_name="c", subcore_axis_name="s",
                                  num_cores=NC, num_subcores=NS)
    fn = pl.kernel(
        functools.partial(_sc_body, B, D, b_per_w),
        out_type=(
            jax.ShapeDtypeStruct((B,), jnp.float32),
            jax.ShapeDtypeStruct((B * D,), jnp.float32),
            jax.ShapeDtypeStruct((B * D,), jnp.float32),
            jax.ShapeDtypeStruct((B,), jnp.float32),
        ),
        mesh=mesh,
        scratch_types=[
            pltpu.VMEM((b_per_w,), jnp.int32),
            pltpu.VMEM((b_per_w,), jnp.int32),
            pltpu.VMEM((b_per_w,), jnp.int32),
            pltpu.VMEM((b_per_w,), jnp.int32),
            pltpu.VMEM((CHUNK, TS, D), jnp.float32),
            pltpu.VMEM((CHUNK, TS, D), jnp.float32),
            pltpu.VMEM((CHUNK, TS, D), jnp.float32),
            pltpu.VMEM((CHUNK, TS, D), jnp.float32),
            pltpu.VMEM((CHUNK * D,), jnp.float32),
            pltpu.VMEM((CHUNK * D,), jnp.float32),
            pltpu.VMEM((CHUNK * D,), jnp.float32),
            pltpu.VMEM((CHUNK * D,), jnp.float32),
            pltpu.VMEM((CHUNK * D,), jnp.float32),
            pltpu.VMEM((CHUNK * D,), jnp.float32),
            pltpu.VMEM((b_per_w,), jnp.float32),
            pltpu.VMEM((b_per_w,), jnp.float32),
            pltpu.SemaphoreType.DMA,
            pltpu.SemaphoreType.DMA,
            pltpu.SemaphoreType.DMA,
            pltpu.SemaphoreType.DMA,
        ],
        compiler_params=pltpu.CompilerParams(needs_layout_passes=False,
                                             use_tc_tiling_on_sc=True),
        name="emb_icd_sc",
    )
    theta_tiles = theta_table.reshape(-1, TS, D)
    a_tiles = a_table.reshape(-1, TS, D)
    return fn(user_idx, item_idx, know.reshape(-1), theta_tiles, a_tiles,
              b_table.reshape(-1))


def kernel(user_idx, item_idx, know, theta_table, a_table, b_table):
    user_idx = user_idx.astype(jnp.int32)
    item_idx = item_idx.astype(jnp.int32)
    B, D = know.shape
    pred, theta_f, a_f, b_flat = _emb_icd(user_idx, item_idx, know,
                                          theta_table, a_table, b_table)
    return (pred, theta_f.reshape(B, D), a_f.reshape(B, D),
            b_flat.reshape(-1, 1))
